# baseline (speedup 1.0000x reference)
import functools

import jax
import jax.numpy as jnp
from jax.experimental import pallas as pl
from jax.experimental.pallas import tpu as pltpu

_MIB = 1024 * 1024


def _scse_body(x_ref, wsq_ref, wex_ref, wsse_ref, out_ref, *, inv_hw):
    nb, c, h, w = x_ref.shape
    hw = h * w
    x = x_ref[...].reshape(nb, c, hw)

    m = jnp.sum(x, axis=2) * inv_hw
    s = jax.lax.dot_general(m, wsq_ref[...], (((1,), (1,)), ((), ())),
                            preferred_element_type=jnp.float32)
    e = jax.lax.dot_general(s, wex_ref[...], (((1,), (1,)), ((), ())),
                            preferred_element_type=jnp.float32)
    gate_c = jax.nn.sigmoid(e)

    w_row = wsse_ref[...]
    q = jnp.concatenate(
        [jnp.dot(w_row, x[i], preferred_element_type=jnp.float32)
         for i in range(nb)], axis=0)
    gate_s = jax.nn.sigmoid(q)

    out = x * (gate_c[:, :, None] + gate_s[:, None, :])
    out_ref[...] = out.reshape(nb, c, h, w)


def kernel(u_nchw, w_sq, w_ex, w_sse):
    B, C, H, W = u_nchw.shape
    bblk = 2
    while bblk > 1 and B % bblk:
        bblk //= 2
    wsse_row = w_sse.reshape(1, C)

    return pl.pallas_call(
        functools.partial(_scse_body, inv_hw=1.0 / (H * W)),
        out_shape=jax.ShapeDtypeStruct((B, C, H, W), u_nchw.dtype),
        grid=(B // bblk,),
        in_specs=[
            pl.BlockSpec((bblk, C, H, W), lambda b: (b, 0, 0, 0)),
            pl.BlockSpec((C // 2, C), lambda b: (0, 0)),
            pl.BlockSpec((C, C // 2), lambda b: (0, 0)),
            pl.BlockSpec((1, C), lambda b: (0, 0)),
        ],
        out_specs=pl.BlockSpec((bblk, C, H, W), lambda b: (b, 0, 0, 0)),
        compiler_params=pltpu.CompilerParams(
            dimension_semantics=("parallel",),
            vmem_limit_bytes=56 * _MIB,
        ),
        cost_estimate=pl.CostEstimate(
            flops=6 * B * C * H * W,
            transcendentals=B * (H * W + C),
            bytes_accessed=2 * B * C * H * W * u_nchw.dtype.itemsize,
        ),
    )(u_nchw, w_sq, w_ex, wsse_row)


# speedup vs baseline: 3.7750x; 3.7750x over previous
import functools

import jax
import jax.numpy as jnp
from jax.experimental import pallas as pl
from jax.experimental.pallas import tpu as pltpu

_MIB = 1024 * 1024


def _scse_body(x_ref, wsq_ref, wex_ref, wsse_ref, out_ref, *, inv_hw):
    x = x_ref[...]
    nb = x.shape[0]

    m = jnp.sum(x, axis=2) * inv_hw
    s = jax.lax.dot_general(m, wsq_ref[...], (((1,), (1,)), ((), ())),
                            preferred_element_type=jnp.float32)
    e = jax.lax.dot_general(s, wex_ref[...], (((1,), (1,)), ((), ())),
                            preferred_element_type=jnp.float32)
    gate_c = jax.nn.sigmoid(e)

    w_row = wsse_ref[...]
    q = jnp.concatenate(
        [jnp.dot(w_row, x[i], preferred_element_type=jnp.float32)
         for i in range(nb)], axis=0)
    gate_s = jax.nn.sigmoid(q)

    out_ref[...] = x * (gate_c[:, :, None] + gate_s[:, None, :])


def kernel(u_nchw, w_sq, w_ex, w_sse):
    B, C, H, W = u_nchw.shape
    HW = H * W
    x = u_nchw.reshape(B, C, HW)
    bblk = 8
    while bblk > 1 and B % bblk:
        bblk //= 2
    wsse_row = w_sse.reshape(1, C)

    out = pl.pallas_call(
        functools.partial(_scse_body, inv_hw=1.0 / HW),
        out_shape=jax.ShapeDtypeStruct((B, C, HW), x.dtype),
        grid=(B // bblk,),
        in_specs=[
            pl.BlockSpec((bblk, C, HW), lambda b: (b, 0, 0)),
            pl.BlockSpec((C // 2, C), lambda b: (0, 0)),
            pl.BlockSpec((C, C // 2), lambda b: (0, 0)),
            pl.BlockSpec((1, C), lambda b: (0, 0)),
        ],
        out_specs=pl.BlockSpec((bblk, C, HW), lambda b: (b, 0, 0)),
        compiler_params=pltpu.CompilerParams(
            dimension_semantics=("parallel",),
            vmem_limit_bytes=56 * _MIB,
        ),
        cost_estimate=pl.CostEstimate(
            flops=6 * B * C * HW,
            transcendentals=B * (HW + C),
            bytes_accessed=2 * B * C * HW * x.dtype.itemsize,
        ),
    )(x, w_sq, w_ex, wsse_row)
    return out.reshape(B, C, H, W)
